# baseline (speedup 1.0000x reference)
import functools

import jax
import jax.numpy as jnp
from jax import lax
from jax.experimental import pallas as pl
from jax.experimental.pallas import tpu as pltpu
from jax.experimental.pallas import tpu_sc as plsc

B = 4096
F = 26
V = 100000
D = 16
NCONT = 13
H = 400
INC = F * D

_BN_INV = 1.0 / (1.0 + 1e-5) ** 0.5


@functools.cache
def _sc_gather_fn():
    mesh = plsc.VectorSubcoreMesh(core_axis_name="c", subcore_axis_name="s",
                                  num_cores=2, num_subcores=16)

    @functools.partial(
        pl.kernel,
        out_type=(
            jax.ShapeDtypeStruct((INC, B), jnp.float32),
            jax.ShapeDtypeStruct((F, B), jnp.float32),
        ),
        mesh=mesh,
        scratch_types=[
            pltpu.VMEM((B,), jnp.int32),
            pltpu.VMEM((D, B), jnp.float32),
            pltpu.VMEM((B,), jnp.float32),
            pltpu.SemaphoreType.DMA,
            pltpu.SemaphoreType.DMA,
        ],
        compiler_params=pltpu.CompilerParams(use_tc_tiling_on_sc=False),
    )
    def _sc_gather(idx_hbm, w2_hbm, w1_hbm, out2, out1,
                   idx_v, val_v, w1_v, sem2, sem1):
        wid = lax.axis_index("s") * 2 + lax.axis_index("c")
        for f in range(F):
            @pl.when(wid == f)
            def _(f=f):
                pltpu.sync_copy(idx_hbm.at[f], idx_v)
                cps = [pltpu.async_copy(w2_hbm.at[f, d].at[idx_v],
                                        val_v.at[d], sem2)
                       for d in range(D)]
                cp1 = pltpu.async_copy(w1_hbm.at[f].at[idx_v], w1_v, sem1)
                for cp in cps:
                    cp.wait()
                cp1.wait()
                pltpu.sync_copy(val_v, out2.at[pl.ds(f * D, D)])
                pltpu.sync_copy(w1_v, out1.at[f])

    return _sc_gather


BT = 512
GRID = B // BT


def _tc_body(sec_ref, first_ref, cont_ref, ssel_ref, et_ref, w1c_ref, bias_ref,
             wd0a_ref, wd0b_ref, bd0_ref, wd1_ref, bd1_ref, wd2_ref, o_ref):
    f32 = jnp.float32
    nn = lambda a, b: lax.dot_general(a, b, (((1,), (0,)), ((), ())),
                                      preferred_element_type=f32)
    sec = sec_ref[...]
    cont = cont_ref[...]
    sec_cont = nn(et_ref[...], cont)
    s = nn(ssel_ref[...][:, :INC], sec) + nn(ssel_ref[...][:, :NCONT * D],
                                             sec_cont)
    sumsq = (jnp.sum(sec * sec, axis=0, keepdims=True)
             + jnp.sum(sec_cont * sec_cont, axis=0, keepdims=True))
    fm_second = 0.5 * (jnp.sum(s * s, axis=0, keepdims=True) - sumsq)
    fm_first = (jnp.sum(first_ref[...], axis=0, keepdims=True)
                + nn(w1c_ref[...], cont))
    h = nn(wd0a_ref[...], sec) + nn(wd0b_ref[...], sec_cont) + bd0_ref[...]
    h = jnp.maximum(h, 0.0)
    h = jnp.maximum(nn(wd1_ref[...], h) + bd1_ref[...], 0.0)
    deep_out = nn(wd2_ref[...][:, :H], h)
    logit = wd2_ref[0, H] + fm_first + fm_second + deep_out
    p = jax.nn.sigmoid(logit)
    o_ref[...] = jnp.concatenate([1.0 - p, p], axis=0)


def _tc_call(sec_t, first_t, cont_t, ssel, et, w1c, bias2,
             wd0a, wd0b, bd0c, wd1, bd1c, wd2b):
    wspec = lambda a: pl.BlockSpec(a.shape, lambda i: (0,) * a.ndim)
    specs = [
        pl.BlockSpec((INC, BT), lambda i: (0, i)),
        pl.BlockSpec((F, BT), lambda i: (0, i)),
        pl.BlockSpec((NCONT, BT), lambda i: (0, i)),
    ] + [wspec(a) for a in (ssel, et, w1c, bias2, wd0a, wd0b, bd0c,
                            wd1, bd1c, wd2b)]
    return pl.pallas_call(
        _tc_body,
        grid=(GRID,),
        in_specs=specs,
        out_specs=pl.BlockSpec((2, BT), lambda i: (0, i)),
        out_shape=jax.ShapeDtypeStruct((2, B), jnp.float32),
        compiler_params=pltpu.CompilerParams(
            dimension_semantics=("arbitrary",)),
    )(sec_t, first_t, cont_t, ssel, et, w1c, bias2, wd0a, wd0b, bd0c,
      wd1, bd1c, wd2b)


def kernel(cat_feats, cont_feats, bias, W1_cat, W1_cont, W2_cat, W2_cont,
           Wd0, bd0, g0, be0, Wd1, bd1, g1, be1, Wd2, bd2):
    f32 = jnp.float32
    idx_t = cat_feats.astype(jnp.int32).T
    cont_t = cont_feats.astype(f32).T
    w2_t = jnp.transpose(W2_cat, (0, 2, 1))
    w1_t = W1_cat.reshape(F, V)
    sec_t, first_t = _sc_gather_fn()(idx_t, w2_t, w1_t)
    ssel = jnp.tile(jnp.eye(D, dtype=f32), (1, F + NCONT))
    et = (jnp.eye(NCONT, dtype=f32)[:, :, None]
          * W2_cont[None, :, :]).reshape(NCONT, NCONT * D).T
    s0 = (_BN_INV * g0).astype(f32)
    s1 = (_BN_INV * g1).astype(f32)
    wd0f = Wd0 * s0[:, None]
    bd0f = (bd0 * s0 + be0)[:, None]
    wd1f = Wd1 * s1[:, None]
    bd1f = (bd1 * s1 + be1)[:, None]
    wd2b = jnp.concatenate(
        [Wd2, (bias + bd2).reshape(1, 1)], axis=1)
    out_t = _tc_call(
        sec_t, first_t, cont_t, ssel, et,
        W1_cont.reshape(1, NCONT).astype(f32), bias.reshape(1, 1).astype(f32),
        wd0f[:, :INC], wd0f[:, INC:], bd0f, wd1f, bd1f, wd2b)
    return out_t.T


# speedup vs baseline: 1.0360x; 1.0360x over previous
import functools

import jax
import jax.numpy as jnp
from jax import lax
from jax.experimental import pallas as pl
from jax.experimental.pallas import tpu as pltpu
from jax.experimental.pallas import tpu_sc as plsc

B = 4096
F = 26
V = 100000
D = 16
NCONT = 13
H = 400
INC = F * D
FH = F // 2
HB = B // 2
FD = FH * D

_BN_INV = 1.0 / (1.0 + 1e-5) ** 0.5


@functools.cache
def _sc_gather_fn():
    mesh = plsc.VectorSubcoreMesh(core_axis_name="c", subcore_axis_name="s",
                                  num_cores=2, num_subcores=16)

    @functools.partial(
        pl.kernel,
        out_type=(
            jax.ShapeDtypeStruct((FD, B), jnp.float32),
            jax.ShapeDtypeStruct((FH, B), jnp.float32),
        ),
        mesh=mesh,
        scratch_types=[
            pltpu.VMEM((HB,), jnp.int32),
            pltpu.VMEM((D, HB), jnp.float32),
            pltpu.VMEM((HB,), jnp.float32),
            pltpu.SemaphoreType.DMA,
            pltpu.SemaphoreType.DMA,
        ],
        compiler_params=pltpu.CompilerParams(use_tc_tiling_on_sc=False),
    )
    def _sc_gather(idx_hbm, w2_hbm, w1_hbm, out2, out1,
                   idx_v, val_v, w1_v, sem2, sem1):
        wid = lax.axis_index("s") * 2 + lax.axis_index("c")
        for fl in range(FH):
            for h in range(2):
                @pl.when(wid == fl * 2 + h)
                def _(fl=fl, h=h):
                    pltpu.sync_copy(idx_hbm.at[fl, pl.ds(h * HB, HB)], idx_v)
                    cps = [pltpu.async_copy(w2_hbm.at[fl, d].at[idx_v],
                                            val_v.at[d], sem2)
                           for d in range(D)]
                    cp1 = pltpu.async_copy(w1_hbm.at[fl].at[idx_v], w1_v, sem1)
                    for cp in cps:
                        cp.wait()
                    cp1.wait()
                    pltpu.sync_copy(
                        val_v, out2.at[pl.ds(fl * D, D), pl.ds(h * HB, HB)])
                    pltpu.sync_copy(w1_v, out1.at[fl, pl.ds(h * HB, HB)])

    return _sc_gather


BT = 512
GRID = B // BT


def _tc_body(seca_ref, secb_ref, firsta_ref, firstb_ref, cont_ref,
             ssel_ref, et_ref, w1c_ref,
             wd0a_ref, wd0b_ref, wd0c_ref, bd0_ref, wd1_ref, bd1_ref,
             wd2_ref, o_ref):
    f32 = jnp.float32
    nn = lambda a, b: lax.dot_general(a, b, (((1,), (0,)), ((), ())),
                                      preferred_element_type=f32)
    seca = seca_ref[...]
    secb = secb_ref[...]
    cont = cont_ref[...]
    secc = nn(et_ref[...], cont)
    s = nn(ssel_ref[...], seca + secb + secc)
    sumsq = (jnp.sum(seca * seca, axis=0, keepdims=True)
             + jnp.sum(secb * secb, axis=0, keepdims=True)
             + jnp.sum(secc * secc, axis=0, keepdims=True))
    fm_second = 0.5 * (jnp.sum(s * s, axis=0, keepdims=True) - sumsq)
    fm_first = (jnp.sum(firsta_ref[...], axis=0, keepdims=True)
                + jnp.sum(firstb_ref[...], axis=0, keepdims=True)
                + nn(w1c_ref[...], cont))
    h = (nn(wd0a_ref[...], seca) + nn(wd0b_ref[...], secb)
         + nn(wd0c_ref[...], secc) + bd0_ref[...])
    h = jnp.maximum(h, 0.0)
    h = jnp.maximum(nn(wd1_ref[...], h) + bd1_ref[...], 0.0)
    deep_out = nn(wd2_ref[...][:, :H], h)
    logit = wd2_ref[0, H] + fm_first + fm_second + deep_out
    p = jax.nn.sigmoid(logit)
    o_ref[...] = jnp.concatenate([1.0 - p, p], axis=0)


def _tc_call(seca, secb, firsta, firstb, cont_t, ssel, et, w1c,
             wd0a, wd0b, wd0c, bd0c, wd1, bd1c, wd2b):
    wspec = lambda a: pl.BlockSpec(a.shape, lambda i: (0,) * a.ndim)
    specs = [
        pl.BlockSpec((FD, BT), lambda i: (0, i)),
        pl.BlockSpec((FD, BT), lambda i: (0, i)),
        pl.BlockSpec((FH, BT), lambda i: (0, i)),
        pl.BlockSpec((FH, BT), lambda i: (0, i)),
        pl.BlockSpec((NCONT, BT), lambda i: (0, i)),
    ] + [wspec(a) for a in (ssel, et, w1c, wd0a, wd0b, wd0c, bd0c,
                            wd1, bd1c, wd2b)]
    return pl.pallas_call(
        _tc_body,
        grid=(GRID,),
        in_specs=specs,
        out_specs=pl.BlockSpec((2, BT), lambda i: (0, i)),
        out_shape=jax.ShapeDtypeStruct((2, B), jnp.float32),
        compiler_params=pltpu.CompilerParams(
            dimension_semantics=("arbitrary",)),
    )(seca, secb, firsta, firstb, cont_t, ssel, et, w1c,
      wd0a, wd0b, wd0c, bd0c, wd1, bd1c, wd2b)


def kernel(cat_feats, cont_feats, bias, W1_cat, W1_cont, W2_cat, W2_cont,
           Wd0, bd0, g0, be0, Wd1, bd1, g1, be1, Wd2, bd2):
    f32 = jnp.float32
    idx_t = cat_feats.astype(jnp.int32).T
    cont_t = cont_feats.astype(f32).T
    gather = _sc_gather_fn()
    halves = []
    for hf in range(2):
        f0 = hf * FH
        w2h = jnp.transpose(W2_cat[f0:f0 + FH], (0, 2, 1))
        w1h = W1_cat[f0:f0 + FH].reshape(FH, V)
        halves.append(gather(idx_t[f0:f0 + FH], w2h, w1h))
    (seca, firsta), (secb, firstb) = halves
    ssel = jnp.tile(jnp.eye(D, dtype=f32), (1, FH))
    et = (jnp.eye(NCONT, dtype=f32)[:, :, None]
          * W2_cont[None, :, :]).reshape(NCONT, NCONT * D).T
    s0 = (_BN_INV * g0).astype(f32)
    s1 = (_BN_INV * g1).astype(f32)
    wd0f = Wd0 * s0[:, None]
    bd0f = (bd0 * s0 + be0)[:, None]
    wd1f = Wd1 * s1[:, None]
    bd1f = (bd1 * s1 + be1)[:, None]
    wd2b = jnp.concatenate(
        [Wd2, (bias + bd2).reshape(1, 1)], axis=1)
    out_t = _tc_call(
        seca, secb, firsta, firstb, cont_t, ssel, et,
        W1_cont.reshape(1, NCONT).astype(f32),
        wd0f[:, :FD], wd0f[:, FD:INC], wd0f[:, INC:], bd0f, wd1f, bd1f, wd2b)
    return out_t.T
